# baseline (speedup 1.0000x reference)
import functools

import jax
import jax.numpy as jnp
from jax import lax
from jax.experimental import pallas as pl
from jax.experimental.pallas import tpu as pltpu
from jax.experimental.pallas import tpu_sc as plsc

HIDDEN = 256
INTER = 1024
N_NODES = 10000
N_EDGES = 160000
EPS = 1e-06

_Q = HIDDEN // 4
_N_SLABS = 16
_N_TILES = 16

_CHUNK = 128
_PCH = -(-N_EDGES // (_N_TILES * _CHUNK))
_EPAD = _N_TILES * _PCH * _CHUNK
_PAD_ROWS = 16
_ACC_ROWS = N_NODES + _PAD_ROWS

_RPT = 624
_REM_BASE = _RPT * _N_TILES
_REM = N_NODES - _REM_BASE
_ZR = 208


_LN_BLK = 2000


def _ln_body(x_ref, gam_ref, bet_ref, h_ref, q0_ref, q1_ref, q2_ref, q3_ref):
    x = x_ref[...]
    mu = jnp.mean(x, axis=-1, keepdims=True)
    xc = x - mu
    var = jnp.mean(xc * xc, axis=-1, keepdims=True)
    h = xc * lax.rsqrt(var + EPS) * gam_ref[...] + bet_ref[...]
    h_ref[...] = h
    q0_ref[...] = h[:, 0 * _Q:1 * _Q]
    q1_ref[...] = h[:, 1 * _Q:2 * _Q]
    q2_ref[...] = h[:, 2 * _Q:3 * _Q]
    q3_ref[...] = h[:, 3 * _Q:4 * _Q]


_ln_call = pl.pallas_call(
    _ln_body,
    grid=(N_NODES // _LN_BLK,),
    in_specs=[
        pl.BlockSpec((_LN_BLK, HIDDEN), lambda i: (i, 0)),
        pl.BlockSpec((1, HIDDEN), lambda i: (0, 0)),
        pl.BlockSpec((1, HIDDEN), lambda i: (0, 0)),
    ],
    out_specs=[pl.BlockSpec((_LN_BLK, HIDDEN), lambda i: (i, 0))] +
              [pl.BlockSpec((_LN_BLK, _Q), lambda i: (i, 0))] * 4,
    out_shape=[jax.ShapeDtypeStruct((N_NODES, HIDDEN), jnp.float32)] +
              [jax.ShapeDtypeStruct((N_NODES, _Q), jnp.float32)] * 4,
)


_sc_mesh = plsc.VectorSubcoreMesh(core_axis_name="c", subcore_axis_name="s")


@functools.partial(
    pl.kernel,
    mesh=_sc_mesh,
    out_type=jax.ShapeDtypeStruct((_N_SLABS, N_NODES, _Q), jnp.float32),
    scratch_types=[
        pltpu.VMEM((_PCH, _CHUNK), jnp.int32),
        pltpu.VMEM((_PCH, _CHUNK), jnp.int32),
        pltpu.VMEM((4, _CHUNK, _Q), jnp.float32),
        pltpu.VMEM((_ZR, _Q), jnp.float32),
        pltpu.VMEM_SHARED((_ACC_ROWS, _Q), jnp.float32),
        pltpu.SemaphoreType.DMA,
        pltpu.SemaphoreType.DMA,
    ],
    compiler_params=pltpu.CompilerParams(use_tc_tiling_on_sc=False),
)
def _sc_scatter(q0, q1, q2, q3, src0, dst0, src1, dst1, src2, dst2,
                src3, dst3, zeros_hbm, out_hbm,
                sblk, dblk, rows, zrows_v, accum, sem_g, sem_s):
    cid = lax.axis_index("c")
    sid = lax.axis_index("s")
    pltpu.sync_copy(zeros_hbm, zrows_v)
    quarters = (q0, q1, q2, q3)
    edges = ((src0, dst0), (src1, dst1), (src2, dst2), (src3, dst3))
    for k in range(4):

        @pl.when(cid == k // 2)
        def _edge_set(k=k):
            src2d, dst2d = edges[k]
            row0 = sid * _PCH
            pltpu.sync_copy(src2d.at[pl.ds(row0, _PCH)], sblk)
            pltpu.sync_copy(dst2d.at[pl.ds(row0, _PCH)], dblk)
            for q in range(4):
                s = k * 4 + q
                hq = quarters[q]
                for t in range(_RPT // _ZR):
                    pltpu.sync_copy(
                        zrows_v, accum.at[pl.ds(sid * _RPT + t * _ZR, _ZR)])

                @pl.when(sid == 0)
                def _zrem():
                    pltpu.sync_copy(zrows_v.at[pl.ds(0, _REM)],
                                    accum.at[pl.ds(_REM_BASE, _REM)])

                plsc.subcore_barrier()

                pltpu.async_copy(hq.at[sblk.at[0]], rows.at[0], sem_g)
                pltpu.async_copy(hq.at[sblk.at[1]], rows.at[1], sem_g)

                def body(j, carry):
                    b = lax.rem(j, 4)
                    pltpu.make_async_copy(hq.at[sblk.at[j]],
                                          rows.at[b], sem_g).wait()

                    @pl.when(j >= 2)
                    def _wait_s():
                        jp = jnp.maximum(j - 2, 0)
                        pltpu.make_async_copy(rows.at[lax.rem(jp, 4)],
                                              accum.at[dblk.at[jp]],
                                              sem_s).wait()

                    @pl.when(j < _PCH - 2)
                    def _next():
                        pltpu.async_copy(hq.at[sblk.at[j + 2]],
                                         rows.at[lax.rem(j + 2, 4)], sem_g)

                    pltpu.async_copy(rows.at[b], accum.at[dblk.at[j]],
                                     sem_s, add=True)
                    return carry

                lax.fori_loop(0, _PCH, body, 0)
                pltpu.make_async_copy(rows.at[(_PCH - 2) % 4],
                                      accum.at[dblk.at[_PCH - 2]],
                                      sem_s).wait()
                pltpu.make_async_copy(rows.at[(_PCH - 1) % 4],
                                      accum.at[dblk.at[_PCH - 1]],
                                      sem_s).wait()
                plsc.subcore_barrier()
                pltpu.sync_copy(accum.at[pl.ds(sid * _RPT, _RPT)],
                                out_hbm.at[s, pl.ds(sid * _RPT, _RPT)])

                @pl.when(sid == 0)
                def _wrem():
                    pltpu.sync_copy(accum.at[pl.ds(_REM_BASE, _REM)],
                                    out_hbm.at[s, pl.ds(_REM_BASE, _REM)])


_DN_BLK = 1000


def _dense_body(h_ref, g_ref, wc_ref, w1_ref, b1_ref, w2_ref, b2_ref, o_ref):
    acc = jnp.zeros((_DN_BLK, HIDDEN), jnp.float32)
    for s in range(_N_SLABS):
        acc += jnp.dot(g_ref[s], wc_ref[s], preferred_element_type=jnp.float32)
    h2 = h_ref[...] + acc
    inter = jnp.dot(h2, w1_ref[...], preferred_element_type=jnp.float32)
    inter = inter + b1_ref[...]
    inter = jnp.where(inter >= 0, inter, 0.01 * inter)
    ff = jnp.dot(inter, w2_ref[...], preferred_element_type=jnp.float32)
    o_ref[...] = h2 + ff + b2_ref[...]


_dense_call = pl.pallas_call(
    _dense_body,
    grid=(N_NODES // _DN_BLK,),
    in_specs=[
        pl.BlockSpec((_DN_BLK, HIDDEN), lambda i: (i, 0)),
        pl.BlockSpec((_N_SLABS, _DN_BLK, _Q), lambda i: (0, i, 0)),
        pl.BlockSpec((_N_SLABS, _Q, HIDDEN), lambda i: (0, 0, 0)),
        pl.BlockSpec((HIDDEN, INTER), lambda i: (0, 0)),
        pl.BlockSpec((1, INTER), lambda i: (0, 0)),
        pl.BlockSpec((INTER, HIDDEN), lambda i: (0, 0)),
        pl.BlockSpec((1, HIDDEN), lambda i: (0, 0)),
    ],
    out_specs=pl.BlockSpec((_DN_BLK, HIDDEN), lambda i: (i, 0)),
    out_shape=jax.ShapeDtypeStruct((N_NODES, HIDDEN), jnp.float32),
)


def kernel(hidden_states, edge_index_i, edge_index_ii, edge_index_iii,
           edge_index_a, W_i, W_ii, W_iii, W_a, ln_gamma, ln_beta,
           ff_w1, ff_b1, ff_w2, ff_b2):
    h, q0, q1, q2, q3 = _ln_call(hidden_states,
                                 ln_gamma.reshape(1, HIDDEN),
                                 ln_beta.reshape(1, HIDDEN))
    npad = _EPAD - N_EDGES
    pad_src = jnp.zeros((npad,), jnp.int32)
    pad_dst = N_NODES + (jnp.arange(npad, dtype=jnp.int32) % _PAD_ROWS)
    er = []
    for e in (edge_index_i, edge_index_ii, edge_index_iii, edge_index_a):
        e32 = e.astype(jnp.int32)
        er += [jnp.concatenate([e32[0], pad_src]).reshape(-1, _CHUNK),
               jnp.concatenate([e32[1], pad_dst]).reshape(-1, _CHUNK)]
    zeros = jnp.zeros((_ZR, _Q), jnp.float32)
    g = _sc_scatter(q0, q1, q2, q3, *er, zeros)
    wc = jnp.stack([W[i * _Q:(i + 1) * _Q]
                    for W in (W_i, W_ii, W_iii, W_a)
                    for i in range(4)])
    return _dense_call(h, g, wc,
                       ff_w1, ff_b1.reshape(1, INTER),
                       ff_w2, ff_b2.reshape(1, HIDDEN))


# speedup vs baseline: 1.5974x; 1.5974x over previous
import functools

import jax
import jax.numpy as jnp
from jax import lax
from jax.experimental import pallas as pl
from jax.experimental.pallas import tpu as pltpu
from jax.experimental.pallas import tpu_sc as plsc

HIDDEN = 256
INTER = 1024
N_NODES = 10000
N_EDGES = 160000
EPS = 1e-06

_Q = HIDDEN // 4
_N_SLABS = 16
_N_TILES = 16

_CHUNK = 80
_PCH = -(-N_EDGES // (_N_TILES * _CHUNK))
_EPAD = _N_TILES * _PCH * _CHUNK
_PAD_ROWS = 16
_ACC_ROWS = N_NODES + _PAD_ROWS

_RPT = 624
_REM_BASE = _RPT * _N_TILES
_REM = N_NODES - _REM_BASE
_ZR = 208


_LN_BLK = 2000


def _ln_body(x_ref, gam_ref, bet_ref, h_ref, q0_ref, q1_ref, q2_ref, q3_ref):
    x = x_ref[...]
    mu = jnp.mean(x, axis=-1, keepdims=True)
    xc = x - mu
    var = jnp.mean(xc * xc, axis=-1, keepdims=True)
    h = xc * lax.rsqrt(var + EPS) * gam_ref[...] + bet_ref[...]
    h_ref[...] = h
    q0_ref[...] = h[:, 0 * _Q:1 * _Q]
    q1_ref[...] = h[:, 1 * _Q:2 * _Q]
    q2_ref[...] = h[:, 2 * _Q:3 * _Q]
    q3_ref[...] = h[:, 3 * _Q:4 * _Q]


_ln_call = pl.pallas_call(
    _ln_body,
    grid=(N_NODES // _LN_BLK,),
    in_specs=[
        pl.BlockSpec((_LN_BLK, HIDDEN), lambda i: (i, 0)),
        pl.BlockSpec((1, HIDDEN), lambda i: (0, 0)),
        pl.BlockSpec((1, HIDDEN), lambda i: (0, 0)),
    ],
    out_specs=[pl.BlockSpec((_LN_BLK, HIDDEN), lambda i: (i, 0))] +
              [pl.BlockSpec((_LN_BLK, _Q), lambda i: (i, 0))] * 4,
    out_shape=[jax.ShapeDtypeStruct((N_NODES, HIDDEN), jnp.float32)] +
              [jax.ShapeDtypeStruct((N_NODES, _Q), jnp.float32)] * 4,
)


_sc_mesh = plsc.VectorSubcoreMesh(core_axis_name="c", subcore_axis_name="s")


@functools.partial(
    pl.kernel,
    mesh=_sc_mesh,
    out_type=jax.ShapeDtypeStruct((_N_SLABS, N_NODES, _Q), jnp.float32),
    scratch_types=[
        pltpu.VMEM((_PCH, _CHUNK), jnp.int32),
        pltpu.VMEM((_PCH, _CHUNK), jnp.int32),
        pltpu.VMEM((4, _CHUNK, _Q), jnp.float32),
        pltpu.VMEM((_ZR, _Q), jnp.float32),
        pltpu.VMEM_SHARED((_ACC_ROWS, _Q), jnp.float32),
        pltpu.SemaphoreType.DMA,
        pltpu.SemaphoreType.DMA,
    ],
    compiler_params=pltpu.CompilerParams(use_tc_tiling_on_sc=False),
)
def _sc_scatter(q0, q1, q2, q3, src0, dst0, src1, dst1, src2, dst2,
                src3, dst3, zeros_hbm, out_hbm,
                sblk, dblk, rows, zrows_v, accum, sem_g, sem_s):
    cid = lax.axis_index("c")
    sid = lax.axis_index("s")
    pltpu.sync_copy(zeros_hbm, zrows_v)
    quarters = (q0, q1, q2, q3)
    edges = ((src0, dst0), (src1, dst1), (src2, dst2), (src3, dst3))
    for k in range(4):

        @pl.when(cid == k // 2)
        def _edge_set(k=k):
            src2d, dst2d = edges[k]
            row0 = sid * _PCH
            pltpu.sync_copy(src2d.at[pl.ds(row0, _PCH)], sblk)
            pltpu.sync_copy(dst2d.at[pl.ds(row0, _PCH)], dblk)
            for q in range(4):
                s = k * 4 + q
                hq = quarters[q]
                for t in range(_RPT // _ZR):
                    pltpu.sync_copy(
                        zrows_v, accum.at[pl.ds(sid * _RPT + t * _ZR, _ZR)])

                @pl.when(sid == 0)
                def _zrem():
                    pltpu.sync_copy(zrows_v.at[pl.ds(0, _REM)],
                                    accum.at[pl.ds(_REM_BASE, _REM)])

                plsc.subcore_barrier()

                pltpu.async_copy(hq.at[sblk.at[0]], rows.at[0], sem_g)
                pltpu.async_copy(hq.at[sblk.at[1]], rows.at[1], sem_g)

                def body(j, carry):
                    b = lax.rem(j, 4)
                    pltpu.make_async_copy(hq.at[sblk.at[j]],
                                          rows.at[b], sem_g).wait()

                    @pl.when(j >= 2)
                    def _wait_s():
                        jp = jnp.maximum(j - 2, 0)
                        pltpu.make_async_copy(rows.at[lax.rem(jp, 4)],
                                              accum.at[dblk.at[jp]],
                                              sem_s).wait()

                    @pl.when(j < _PCH - 2)
                    def _next():
                        pltpu.async_copy(hq.at[sblk.at[j + 2]],
                                         rows.at[lax.rem(j + 2, 4)], sem_g)

                    pltpu.async_copy(rows.at[b], accum.at[dblk.at[j]],
                                     sem_s, add=True)
                    return carry

                lax.fori_loop(0, _PCH, body, 0)
                pltpu.make_async_copy(rows.at[(_PCH - 2) % 4],
                                      accum.at[dblk.at[_PCH - 2]],
                                      sem_s).wait()
                pltpu.make_async_copy(rows.at[(_PCH - 1) % 4],
                                      accum.at[dblk.at[_PCH - 1]],
                                      sem_s).wait()
                plsc.subcore_barrier()
                pltpu.sync_copy(accum.at[pl.ds(sid * _RPT, _RPT)],
                                out_hbm.at[s, pl.ds(sid * _RPT, _RPT)])

                @pl.when(sid == 0)
                def _wrem():
                    pltpu.sync_copy(accum.at[pl.ds(_REM_BASE, _REM)],
                                    out_hbm.at[s, pl.ds(_REM_BASE, _REM)])


_DN_BLK = 1000


def _dense_body(h_ref, g_ref, wc_ref, w1_ref, b1_ref, w2_ref, b2_ref, o_ref):
    acc = jnp.zeros((_DN_BLK, HIDDEN), jnp.float32)
    for s in range(_N_SLABS):
        acc += jnp.dot(g_ref[s], wc_ref[s], preferred_element_type=jnp.float32)
    h2 = h_ref[...] + acc
    inter = jnp.dot(h2, w1_ref[...], preferred_element_type=jnp.float32)
    inter = inter + b1_ref[...]
    inter = jnp.where(inter >= 0, inter, 0.01 * inter)
    ff = jnp.dot(inter, w2_ref[...], preferred_element_type=jnp.float32)
    o_ref[...] = h2 + ff + b2_ref[...]


_dense_call = pl.pallas_call(
    _dense_body,
    grid=(N_NODES // _DN_BLK,),
    in_specs=[
        pl.BlockSpec((_DN_BLK, HIDDEN), lambda i: (i, 0)),
        pl.BlockSpec((_N_SLABS, _DN_BLK, _Q), lambda i: (0, i, 0)),
        pl.BlockSpec((_N_SLABS, _Q, HIDDEN), lambda i: (0, 0, 0)),
        pl.BlockSpec((HIDDEN, INTER), lambda i: (0, 0)),
        pl.BlockSpec((1, INTER), lambda i: (0, 0)),
        pl.BlockSpec((INTER, HIDDEN), lambda i: (0, 0)),
        pl.BlockSpec((1, HIDDEN), lambda i: (0, 0)),
    ],
    out_specs=pl.BlockSpec((_DN_BLK, HIDDEN), lambda i: (i, 0)),
    out_shape=jax.ShapeDtypeStruct((N_NODES, HIDDEN), jnp.float32),
)


def kernel(hidden_states, edge_index_i, edge_index_ii, edge_index_iii,
           edge_index_a, W_i, W_ii, W_iii, W_a, ln_gamma, ln_beta,
           ff_w1, ff_b1, ff_w2, ff_b2):
    h, q0, q1, q2, q3 = _ln_call(hidden_states,
                                 ln_gamma.reshape(1, HIDDEN),
                                 ln_beta.reshape(1, HIDDEN))
    npad = _EPAD - N_EDGES
    pad_src = jnp.zeros((npad,), jnp.int32)
    pad_dst = N_NODES + (jnp.arange(npad, dtype=jnp.int32) % _PAD_ROWS)
    er = []
    for e in (edge_index_i, edge_index_ii, edge_index_iii, edge_index_a):
        e32 = e.astype(jnp.int32)
        er += [jnp.concatenate([e32[0], pad_src]).reshape(-1, _CHUNK),
               jnp.concatenate([e32[1], pad_dst]).reshape(-1, _CHUNK)]
    zeros = jnp.zeros((_ZR, _Q), jnp.float32)
    g = _sc_scatter(q0, q1, q2, q3, *er, zeros)
    wc = jnp.stack([W[i * _Q:(i + 1) * _Q]
                    for W in (W_i, W_ii, W_iii, W_a)
                    for i in range(4)])
    return _dense_call(h, g, wc,
                       ff_w1, ff_b1.reshape(1, INTER),
                       ff_w2, ff_b2.reshape(1, HIDDEN))


# speedup vs baseline: 1.8833x; 1.1790x over previous
import functools

import jax
import jax.numpy as jnp
from jax import lax
from jax.experimental import pallas as pl
from jax.experimental.pallas import tpu as pltpu
from jax.experimental.pallas import tpu_sc as plsc

HIDDEN = 256
INTER = 1024
N_NODES = 10000
N_EDGES = 160000
EPS = 1e-06

_Q = HIDDEN // 4
_N_SLABS = 16
_N_TILES = 16

_CHUNK = 80
_PCH = -(-N_EDGES // (_N_TILES * _CHUNK))
_EPAD = _N_TILES * _PCH * _CHUNK
_PAD_ROWS = 16
_ACC_ROWS = N_NODES + _PAD_ROWS

_RPT = 624
_REM_BASE = _RPT * _N_TILES
_REM = N_NODES - _REM_BASE
_ZR = 208


_LN_BLK = 2000


def _ln_body(x_ref, gam_ref, bet_ref, h_ref, q0_ref, q1_ref, q2_ref, q3_ref):
    x = x_ref[...]
    mu = jnp.mean(x, axis=-1, keepdims=True)
    xc = x - mu
    var = jnp.mean(xc * xc, axis=-1, keepdims=True)
    h = xc * lax.rsqrt(var + EPS) * gam_ref[...] + bet_ref[...]
    h_ref[...] = h
    q0_ref[...] = h[:, 0 * _Q:1 * _Q]
    q1_ref[...] = h[:, 1 * _Q:2 * _Q]
    q2_ref[...] = h[:, 2 * _Q:3 * _Q]
    q3_ref[...] = h[:, 3 * _Q:4 * _Q]


_ln_call = pl.pallas_call(
    _ln_body,
    grid=(N_NODES // _LN_BLK,),
    in_specs=[
        pl.BlockSpec((_LN_BLK, HIDDEN), lambda i: (i, 0)),
        pl.BlockSpec((1, HIDDEN), lambda i: (0, 0)),
        pl.BlockSpec((1, HIDDEN), lambda i: (0, 0)),
    ],
    out_specs=[pl.BlockSpec((_LN_BLK, HIDDEN), lambda i: (i, 0))] +
              [pl.BlockSpec((_LN_BLK, _Q), lambda i: (i, 0))] * 4,
    out_shape=[jax.ShapeDtypeStruct((N_NODES, HIDDEN), jnp.float32)] +
              [jax.ShapeDtypeStruct((N_NODES, _Q), jnp.float32)] * 4,
)


_sc_mesh = plsc.VectorSubcoreMesh(core_axis_name="c", subcore_axis_name="s")


@functools.partial(
    pl.kernel,
    mesh=_sc_mesh,
    out_type=jax.ShapeDtypeStruct((_N_SLABS, N_NODES, _Q), jnp.float32),
    scratch_types=[
        pltpu.VMEM((_PCH, _CHUNK), jnp.int32),
        pltpu.VMEM((_PCH, _CHUNK), jnp.int32),
        pltpu.VMEM((6, _CHUNK, _Q), jnp.float32),
        pltpu.VMEM((_ZR, _Q), jnp.float32),
        pltpu.VMEM_SHARED((_ACC_ROWS, _Q), jnp.float32),
        pltpu.SemaphoreType.DMA,
        pltpu.SemaphoreType.DMA,
    ],
    compiler_params=pltpu.CompilerParams(use_tc_tiling_on_sc=False),
)
def _sc_scatter(q0, q1, q2, q3, src0, dst0, src1, dst1, src2, dst2,
                src3, dst3, zeros_hbm, out_hbm,
                sblk, dblk, rows, zrows_v, accum, sem_g, sem_s):
    cid = lax.axis_index("c")
    sid = lax.axis_index("s")
    pltpu.sync_copy(zeros_hbm, zrows_v)
    quarters = (q0, q1, q2, q3)
    edges = ((src0, dst0), (src1, dst1), (src2, dst2), (src3, dst3))
    for k in range(4):

        @pl.when(cid == k // 2)
        def _edge_set(k=k):
            src2d, dst2d = edges[k]
            row0 = sid * _PCH
            pltpu.sync_copy(src2d.at[pl.ds(row0, _PCH)], sblk)
            pltpu.sync_copy(dst2d.at[pl.ds(row0, _PCH)], dblk)
            for q in range(4):
                s = k * 4 + q
                hq = quarters[q]
                for t in range(_RPT // _ZR):
                    pltpu.sync_copy(
                        zrows_v, accum.at[pl.ds(sid * _RPT + t * _ZR, _ZR)])

                @pl.when(sid == 0)
                def _zrem():
                    pltpu.sync_copy(zrows_v.at[pl.ds(0, _REM)],
                                    accum.at[pl.ds(_REM_BASE, _REM)])

                plsc.subcore_barrier()

                pltpu.async_copy(hq.at[sblk.at[0]], rows.at[0], sem_g)
                pltpu.async_copy(hq.at[sblk.at[1]], rows.at[1], sem_g)
                pltpu.async_copy(hq.at[sblk.at[2]], rows.at[2], sem_g)

                def body(j, carry):
                    b = lax.rem(j, 6)
                    pltpu.make_async_copy(hq.at[sblk.at[j]],
                                          rows.at[b], sem_g).wait()

                    @pl.when(j >= 2)
                    def _wait_s():
                        jp = jnp.maximum(j - 2, 0)
                        pltpu.make_async_copy(rows.at[lax.rem(jp, 6)],
                                              accum.at[dblk.at[jp]],
                                              sem_s).wait()

                    @pl.when(j < _PCH - 3)
                    def _next():
                        pltpu.async_copy(hq.at[sblk.at[j + 3]],
                                         rows.at[lax.rem(j + 3, 6)], sem_g)

                    pltpu.async_copy(rows.at[b], accum.at[dblk.at[j]],
                                     sem_s, add=True)
                    return carry

                lax.fori_loop(0, _PCH, body, 0)
                pltpu.make_async_copy(rows.at[(_PCH - 2) % 6],
                                      accum.at[dblk.at[_PCH - 2]],
                                      sem_s).wait()
                pltpu.make_async_copy(rows.at[(_PCH - 1) % 6],
                                      accum.at[dblk.at[_PCH - 1]],
                                      sem_s).wait()
                plsc.subcore_barrier()
                pltpu.sync_copy(accum.at[pl.ds(sid * _RPT, _RPT)],
                                out_hbm.at[s, pl.ds(sid * _RPT, _RPT)])

                @pl.when(sid == 0)
                def _wrem():
                    pltpu.sync_copy(accum.at[pl.ds(_REM_BASE, _REM)],
                                    out_hbm.at[s, pl.ds(_REM_BASE, _REM)])


_DN_BLK = 1000


def _dense_body(h_ref, g_ref, wc_ref, w1_ref, b1_ref, w2_ref, b2_ref, o_ref):
    acc = jnp.zeros((_DN_BLK, HIDDEN), jnp.float32)
    for s in range(_N_SLABS):
        acc += jnp.dot(g_ref[s], wc_ref[s], preferred_element_type=jnp.float32)
    h2 = h_ref[...] + acc
    inter = jnp.dot(h2, w1_ref[...], preferred_element_type=jnp.float32)
    inter = inter + b1_ref[...]
    inter = jnp.where(inter >= 0, inter, 0.01 * inter)
    ff = jnp.dot(inter, w2_ref[...], preferred_element_type=jnp.float32)
    o_ref[...] = h2 + ff + b2_ref[...]


_dense_call = pl.pallas_call(
    _dense_body,
    grid=(N_NODES // _DN_BLK,),
    in_specs=[
        pl.BlockSpec((_DN_BLK, HIDDEN), lambda i: (i, 0)),
        pl.BlockSpec((_N_SLABS, _DN_BLK, _Q), lambda i: (0, i, 0)),
        pl.BlockSpec((_N_SLABS, _Q, HIDDEN), lambda i: (0, 0, 0)),
        pl.BlockSpec((HIDDEN, INTER), lambda i: (0, 0)),
        pl.BlockSpec((1, INTER), lambda i: (0, 0)),
        pl.BlockSpec((INTER, HIDDEN), lambda i: (0, 0)),
        pl.BlockSpec((1, HIDDEN), lambda i: (0, 0)),
    ],
    out_specs=pl.BlockSpec((_DN_BLK, HIDDEN), lambda i: (i, 0)),
    out_shape=jax.ShapeDtypeStruct((N_NODES, HIDDEN), jnp.float32),
)


def kernel(hidden_states, edge_index_i, edge_index_ii, edge_index_iii,
           edge_index_a, W_i, W_ii, W_iii, W_a, ln_gamma, ln_beta,
           ff_w1, ff_b1, ff_w2, ff_b2):
    h, q0, q1, q2, q3 = _ln_call(hidden_states,
                                 ln_gamma.reshape(1, HIDDEN),
                                 ln_beta.reshape(1, HIDDEN))
    npad = _EPAD - N_EDGES
    pad_src = jnp.zeros((npad,), jnp.int32)
    pad_dst = N_NODES + (jnp.arange(npad, dtype=jnp.int32) % _PAD_ROWS)
    er = []
    for e in (edge_index_i, edge_index_ii, edge_index_iii, edge_index_a):
        e32 = e.astype(jnp.int32)
        er += [jnp.concatenate([e32[0], pad_src]).reshape(-1, _CHUNK),
               jnp.concatenate([e32[1], pad_dst]).reshape(-1, _CHUNK)]
    zeros = jnp.zeros((_ZR, _Q), jnp.float32)
    g = _sc_scatter(q0, q1, q2, q3, *er, zeros)
    wc = jnp.stack([W[i * _Q:(i + 1) * _Q]
                    for W in (W_i, W_ii, W_iii, W_a)
                    for i in range(4)])
    return _dense_call(h, g, wc,
                       ff_w1, ff_b1.reshape(1, INTER),
                       ff_w2, ff_b2.reshape(1, HIDDEN))


# speedup vs baseline: 1.9970x; 1.0604x over previous
import functools

import jax
import jax.numpy as jnp
from jax import lax
from jax.experimental import pallas as pl
from jax.experimental.pallas import tpu as pltpu
from jax.experimental.pallas import tpu_sc as plsc

HIDDEN = 256
INTER = 1024
N_NODES = 10000
N_EDGES = 160000
EPS = 1e-06

_Q = HIDDEN // 4
_N_SLABS = 16
_N_TILES = 16

_CHUNK = 80
_PCH = -(-N_EDGES // (_N_TILES * _CHUNK))
_EPAD = _N_TILES * _PCH * _CHUNK
_PAD_ROWS = 16
_ACC_ROWS = N_NODES + _PAD_ROWS

_RPT = 624
_REM_BASE = _RPT * _N_TILES
_REM = N_NODES - _REM_BASE
_ZR = 208


_LN_BLK = 2000


def _ln_body(x_ref, gam_ref, bet_ref, h_ref, q0_ref, q1_ref, q2_ref, q3_ref):
    x = x_ref[...]
    mu = jnp.mean(x, axis=-1, keepdims=True)
    xc = x - mu
    var = jnp.mean(xc * xc, axis=-1, keepdims=True)
    h = xc * lax.rsqrt(var + EPS) * gam_ref[...] + bet_ref[...]
    h_ref[...] = h
    q0_ref[...] = h[:, 0 * _Q:1 * _Q]
    q1_ref[...] = h[:, 1 * _Q:2 * _Q]
    q2_ref[...] = h[:, 2 * _Q:3 * _Q]
    q3_ref[...] = h[:, 3 * _Q:4 * _Q]


_ln_call = pl.pallas_call(
    _ln_body,
    grid=(N_NODES // _LN_BLK,),
    in_specs=[
        pl.BlockSpec((_LN_BLK, HIDDEN), lambda i: (i, 0)),
        pl.BlockSpec((1, HIDDEN), lambda i: (0, 0)),
        pl.BlockSpec((1, HIDDEN), lambda i: (0, 0)),
    ],
    out_specs=[pl.BlockSpec((_LN_BLK, HIDDEN), lambda i: (i, 0))] +
              [pl.BlockSpec((_LN_BLK, _Q), lambda i: (i, 0))] * 4,
    out_shape=[jax.ShapeDtypeStruct((N_NODES, HIDDEN), jnp.float32)] +
              [jax.ShapeDtypeStruct((N_NODES, _Q), jnp.float32)] * 4,
)


_sc_mesh = plsc.VectorSubcoreMesh(core_axis_name="c", subcore_axis_name="s")


@functools.partial(
    pl.kernel,
    mesh=_sc_mesh,
    out_type=jax.ShapeDtypeStruct((_N_SLABS, N_NODES, _Q), jnp.float32),
    scratch_types=[
        pltpu.VMEM((_PCH, _CHUNK), jnp.int32),
        pltpu.VMEM((_PCH, _CHUNK), jnp.int32),
        pltpu.VMEM((8, _CHUNK, _Q), jnp.float32),
        pltpu.VMEM((_ZR, _Q), jnp.float32),
        pltpu.VMEM_SHARED((_ACC_ROWS, _Q), jnp.float32),
        pltpu.SemaphoreType.DMA,
        pltpu.SemaphoreType.DMA,
    ],
    compiler_params=pltpu.CompilerParams(use_tc_tiling_on_sc=False),
)
def _sc_scatter(q0, q1, q2, q3, src0, dst0, src1, dst1, src2, dst2,
                src3, dst3, zeros_hbm, out_hbm,
                sblk, dblk, rows, zrows_v, accum, sem_g, sem_s):
    cid = lax.axis_index("c")
    sid = lax.axis_index("s")
    pltpu.sync_copy(zeros_hbm, zrows_v)
    quarters = (q0, q1, q2, q3)
    edges = ((src0, dst0), (src1, dst1), (src2, dst2), (src3, dst3))
    for k in range(4):

        @pl.when(cid == k // 2)
        def _edge_set(k=k):
            src2d, dst2d = edges[k]
            row0 = sid * _PCH
            pltpu.sync_copy(src2d.at[pl.ds(row0, _PCH)], sblk)
            pltpu.sync_copy(dst2d.at[pl.ds(row0, _PCH)], dblk)
            for q in range(4):
                s = k * 4 + q
                hq = quarters[q]
                for t in range(_RPT // _ZR):
                    pltpu.sync_copy(
                        zrows_v, accum.at[pl.ds(sid * _RPT + t * _ZR, _ZR)])

                @pl.when(sid == 0)
                def _zrem():
                    pltpu.sync_copy(zrows_v.at[pl.ds(0, _REM)],
                                    accum.at[pl.ds(_REM_BASE, _REM)])

                plsc.subcore_barrier()

                pltpu.async_copy(hq.at[sblk.at[0]], rows.at[0], sem_g)
                pltpu.async_copy(hq.at[sblk.at[1]], rows.at[1], sem_g)
                pltpu.async_copy(hq.at[sblk.at[2]], rows.at[2], sem_g)
                pltpu.async_copy(hq.at[sblk.at[3]], rows.at[3], sem_g)

                def body(j, carry):
                    b = lax.rem(j, 8)
                    pltpu.make_async_copy(hq.at[sblk.at[j]],
                                          rows.at[b], sem_g).wait()

                    @pl.when(j >= 2)
                    def _wait_s():
                        jp = jnp.maximum(j - 2, 0)
                        pltpu.make_async_copy(rows.at[lax.rem(jp, 8)],
                                              accum.at[dblk.at[jp]],
                                              sem_s).wait()

                    @pl.when(j < _PCH - 4)
                    def _next():
                        pltpu.async_copy(hq.at[sblk.at[j + 4]],
                                         rows.at[lax.rem(j + 4, 8)], sem_g)

                    pltpu.async_copy(rows.at[b], accum.at[dblk.at[j]],
                                     sem_s, add=True)
                    return carry

                lax.fori_loop(0, _PCH, body, 0)
                pltpu.make_async_copy(rows.at[(_PCH - 2) % 8],
                                      accum.at[dblk.at[_PCH - 2]],
                                      sem_s).wait()
                pltpu.make_async_copy(rows.at[(_PCH - 1) % 8],
                                      accum.at[dblk.at[_PCH - 1]],
                                      sem_s).wait()
                plsc.subcore_barrier()
                pltpu.sync_copy(accum.at[pl.ds(sid * _RPT, _RPT)],
                                out_hbm.at[s, pl.ds(sid * _RPT, _RPT)])

                @pl.when(sid == 0)
                def _wrem():
                    pltpu.sync_copy(accum.at[pl.ds(_REM_BASE, _REM)],
                                    out_hbm.at[s, pl.ds(_REM_BASE, _REM)])


_DN_BLK = 1000


def _dense_body(h_ref, g_ref, wc_ref, w1_ref, b1_ref, w2_ref, b2_ref, o_ref):
    acc = jnp.zeros((_DN_BLK, HIDDEN), jnp.float32)
    for s in range(_N_SLABS):
        acc += jnp.dot(g_ref[s], wc_ref[s], preferred_element_type=jnp.float32)
    h2 = h_ref[...] + acc
    inter = jnp.dot(h2, w1_ref[...], preferred_element_type=jnp.float32)
    inter = inter + b1_ref[...]
    inter = jnp.where(inter >= 0, inter, 0.01 * inter)
    ff = jnp.dot(inter, w2_ref[...], preferred_element_type=jnp.float32)
    o_ref[...] = h2 + ff + b2_ref[...]


_dense_call = pl.pallas_call(
    _dense_body,
    grid=(N_NODES // _DN_BLK,),
    in_specs=[
        pl.BlockSpec((_DN_BLK, HIDDEN), lambda i: (i, 0)),
        pl.BlockSpec((_N_SLABS, _DN_BLK, _Q), lambda i: (0, i, 0)),
        pl.BlockSpec((_N_SLABS, _Q, HIDDEN), lambda i: (0, 0, 0)),
        pl.BlockSpec((HIDDEN, INTER), lambda i: (0, 0)),
        pl.BlockSpec((1, INTER), lambda i: (0, 0)),
        pl.BlockSpec((INTER, HIDDEN), lambda i: (0, 0)),
        pl.BlockSpec((1, HIDDEN), lambda i: (0, 0)),
    ],
    out_specs=pl.BlockSpec((_DN_BLK, HIDDEN), lambda i: (i, 0)),
    out_shape=jax.ShapeDtypeStruct((N_NODES, HIDDEN), jnp.float32),
)


def kernel(hidden_states, edge_index_i, edge_index_ii, edge_index_iii,
           edge_index_a, W_i, W_ii, W_iii, W_a, ln_gamma, ln_beta,
           ff_w1, ff_b1, ff_w2, ff_b2):
    h, q0, q1, q2, q3 = _ln_call(hidden_states,
                                 ln_gamma.reshape(1, HIDDEN),
                                 ln_beta.reshape(1, HIDDEN))
    npad = _EPAD - N_EDGES
    pad_src = jnp.zeros((npad,), jnp.int32)
    pad_dst = N_NODES + (jnp.arange(npad, dtype=jnp.int32) % _PAD_ROWS)
    er = []
    for e in (edge_index_i, edge_index_ii, edge_index_iii, edge_index_a):
        e32 = e.astype(jnp.int32)
        er += [jnp.concatenate([e32[0], pad_src]).reshape(-1, _CHUNK),
               jnp.concatenate([e32[1], pad_dst]).reshape(-1, _CHUNK)]
    zeros = jnp.zeros((_ZR, _Q), jnp.float32)
    g = _sc_scatter(q0, q1, q2, q3, *er, zeros)
    wc = jnp.stack([W[i * _Q:(i + 1) * _Q]
                    for W in (W_i, W_ii, W_iii, W_a)
                    for i in range(4)])
    return _dense_call(h, g, wc,
                       ff_w1, ff_b1.reshape(1, INTER),
                       ff_w2, ff_b2.reshape(1, HIDDEN))


# speedup vs baseline: 2.0030x; 1.0030x over previous
import functools

import jax
import jax.numpy as jnp
from jax import lax
from jax.experimental import pallas as pl
from jax.experimental.pallas import tpu as pltpu
from jax.experimental.pallas import tpu_sc as plsc

HIDDEN = 256
INTER = 1024
N_NODES = 10000
N_EDGES = 160000
EPS = 1e-06

_Q = HIDDEN // 4
_N_SLABS = 16
_N_TILES = 16

_CHUNK = 80
_PCH = -(-N_EDGES // (_N_TILES * _CHUNK))
_EPAD = _N_TILES * _PCH * _CHUNK
_PAD_ROWS = 16
_ACC_ROWS = N_NODES + _PAD_ROWS

_RPT = 624
_REM_BASE = _RPT * _N_TILES
_REM = N_NODES - _REM_BASE
_ZR = 208


_LN_BLK = 2000


def _ln_body(x_ref, gam_ref, bet_ref, h_ref, q0_ref, q1_ref, q2_ref, q3_ref):
    x = x_ref[...]
    mu = jnp.mean(x, axis=-1, keepdims=True)
    xc = x - mu
    var = jnp.mean(xc * xc, axis=-1, keepdims=True)
    h = xc * lax.rsqrt(var + EPS) * gam_ref[...] + bet_ref[...]
    h_ref[...] = h
    q0_ref[...] = h[:, 0 * _Q:1 * _Q]
    q1_ref[...] = h[:, 1 * _Q:2 * _Q]
    q2_ref[...] = h[:, 2 * _Q:3 * _Q]
    q3_ref[...] = h[:, 3 * _Q:4 * _Q]


_ln_call = pl.pallas_call(
    _ln_body,
    grid=(N_NODES // _LN_BLK,),
    in_specs=[
        pl.BlockSpec((_LN_BLK, HIDDEN), lambda i: (i, 0)),
        pl.BlockSpec((1, HIDDEN), lambda i: (0, 0)),
        pl.BlockSpec((1, HIDDEN), lambda i: (0, 0)),
    ],
    out_specs=[pl.BlockSpec((_LN_BLK, HIDDEN), lambda i: (i, 0))] +
              [pl.BlockSpec((_LN_BLK, _Q), lambda i: (i, 0))] * 4,
    out_shape=[jax.ShapeDtypeStruct((N_NODES, HIDDEN), jnp.float32)] +
              [jax.ShapeDtypeStruct((N_NODES, _Q), jnp.float32)] * 4,
)


_sc_mesh = plsc.VectorSubcoreMesh(core_axis_name="c", subcore_axis_name="s")


@functools.partial(
    pl.kernel,
    mesh=_sc_mesh,
    out_type=jax.ShapeDtypeStruct((_N_SLABS, N_NODES, _Q), jnp.float32),
    scratch_types=[
        pltpu.VMEM((_PCH, _CHUNK), jnp.int32),
        pltpu.VMEM((_PCH, _CHUNK), jnp.int32),
        pltpu.VMEM((10, _CHUNK, _Q), jnp.float32),
        pltpu.VMEM((_ZR, _Q), jnp.float32),
        pltpu.VMEM_SHARED((_ACC_ROWS, _Q), jnp.float32),
        pltpu.SemaphoreType.DMA,
        pltpu.SemaphoreType.DMA,
    ],
    compiler_params=pltpu.CompilerParams(use_tc_tiling_on_sc=False),
)
def _sc_scatter(q0, q1, q2, q3, src0, dst0, src1, dst1, src2, dst2,
                src3, dst3, zeros_hbm, out_hbm,
                sblk, dblk, rows, zrows_v, accum, sem_g, sem_s):
    cid = lax.axis_index("c")
    sid = lax.axis_index("s")
    pltpu.sync_copy(zeros_hbm, zrows_v)
    quarters = (q0, q1, q2, q3)
    edges = ((src0, dst0), (src1, dst1), (src2, dst2), (src3, dst3))
    for k in range(4):

        @pl.when(cid == k // 2)
        def _edge_set(k=k):
            src2d, dst2d = edges[k]
            row0 = sid * _PCH
            pltpu.sync_copy(src2d.at[pl.ds(row0, _PCH)], sblk)
            pltpu.sync_copy(dst2d.at[pl.ds(row0, _PCH)], dblk)
            for q in range(4):
                s = k * 4 + q
                hq = quarters[q]
                for t in range(_RPT // _ZR):
                    pltpu.sync_copy(
                        zrows_v, accum.at[pl.ds(sid * _RPT + t * _ZR, _ZR)])

                @pl.when(sid == 0)
                def _zrem():
                    pltpu.sync_copy(zrows_v.at[pl.ds(0, _REM)],
                                    accum.at[pl.ds(_REM_BASE, _REM)])

                plsc.subcore_barrier()

                pltpu.async_copy(hq.at[sblk.at[0]], rows.at[0], sem_g)
                pltpu.async_copy(hq.at[sblk.at[1]], rows.at[1], sem_g)
                pltpu.async_copy(hq.at[sblk.at[2]], rows.at[2], sem_g)
                pltpu.async_copy(hq.at[sblk.at[3]], rows.at[3], sem_g)
                pltpu.async_copy(hq.at[sblk.at[4]], rows.at[4], sem_g)

                def body(j, carry):
                    b = lax.rem(j, 10)
                    pltpu.make_async_copy(hq.at[sblk.at[j]],
                                          rows.at[b], sem_g).wait()

                    @pl.when(j >= 2)
                    def _wait_s():
                        jp = jnp.maximum(j - 2, 0)
                        pltpu.make_async_copy(rows.at[lax.rem(jp, 10)],
                                              accum.at[dblk.at[jp]],
                                              sem_s).wait()

                    @pl.when(j < _PCH - 5)
                    def _next():
                        pltpu.async_copy(hq.at[sblk.at[j + 5]],
                                         rows.at[lax.rem(j + 5, 10)], sem_g)

                    pltpu.async_copy(rows.at[b], accum.at[dblk.at[j]],
                                     sem_s, add=True)
                    return carry

                lax.fori_loop(0, _PCH, body, 0)
                pltpu.make_async_copy(rows.at[(_PCH - 2) % 10],
                                      accum.at[dblk.at[_PCH - 2]],
                                      sem_s).wait()
                pltpu.make_async_copy(rows.at[(_PCH - 1) % 10],
                                      accum.at[dblk.at[_PCH - 1]],
                                      sem_s).wait()
                plsc.subcore_barrier()
                pltpu.sync_copy(accum.at[pl.ds(sid * _RPT, _RPT)],
                                out_hbm.at[s, pl.ds(sid * _RPT, _RPT)])

                @pl.when(sid == 0)
                def _wrem():
                    pltpu.sync_copy(accum.at[pl.ds(_REM_BASE, _REM)],
                                    out_hbm.at[s, pl.ds(_REM_BASE, _REM)])


_DN_BLK = 1000


def _dense_body(h_ref, g_ref, wc_ref, w1_ref, b1_ref, w2_ref, b2_ref, o_ref):
    acc = jnp.zeros((_DN_BLK, HIDDEN), jnp.float32)
    for s in range(_N_SLABS):
        acc += jnp.dot(g_ref[s], wc_ref[s], preferred_element_type=jnp.float32)
    h2 = h_ref[...] + acc
    inter = jnp.dot(h2, w1_ref[...], preferred_element_type=jnp.float32)
    inter = inter + b1_ref[...]
    inter = jnp.where(inter >= 0, inter, 0.01 * inter)
    ff = jnp.dot(inter, w2_ref[...], preferred_element_type=jnp.float32)
    o_ref[...] = h2 + ff + b2_ref[...]


_dense_call = pl.pallas_call(
    _dense_body,
    grid=(N_NODES // _DN_BLK,),
    in_specs=[
        pl.BlockSpec((_DN_BLK, HIDDEN), lambda i: (i, 0)),
        pl.BlockSpec((_N_SLABS, _DN_BLK, _Q), lambda i: (0, i, 0)),
        pl.BlockSpec((_N_SLABS, _Q, HIDDEN), lambda i: (0, 0, 0)),
        pl.BlockSpec((HIDDEN, INTER), lambda i: (0, 0)),
        pl.BlockSpec((1, INTER), lambda i: (0, 0)),
        pl.BlockSpec((INTER, HIDDEN), lambda i: (0, 0)),
        pl.BlockSpec((1, HIDDEN), lambda i: (0, 0)),
    ],
    out_specs=pl.BlockSpec((_DN_BLK, HIDDEN), lambda i: (i, 0)),
    out_shape=jax.ShapeDtypeStruct((N_NODES, HIDDEN), jnp.float32),
)


def kernel(hidden_states, edge_index_i, edge_index_ii, edge_index_iii,
           edge_index_a, W_i, W_ii, W_iii, W_a, ln_gamma, ln_beta,
           ff_w1, ff_b1, ff_w2, ff_b2):
    h, q0, q1, q2, q3 = _ln_call(hidden_states,
                                 ln_gamma.reshape(1, HIDDEN),
                                 ln_beta.reshape(1, HIDDEN))
    npad = _EPAD - N_EDGES
    pad_src = jnp.zeros((npad,), jnp.int32)
    pad_dst = N_NODES + (jnp.arange(npad, dtype=jnp.int32) % _PAD_ROWS)
    er = []
    for e in (edge_index_i, edge_index_ii, edge_index_iii, edge_index_a):
        e32 = e.astype(jnp.int32)
        er += [jnp.concatenate([e32[0], pad_src]).reshape(-1, _CHUNK),
               jnp.concatenate([e32[1], pad_dst]).reshape(-1, _CHUNK)]
    zeros = jnp.zeros((_ZR, _Q), jnp.float32)
    g = _sc_scatter(q0, q1, q2, q3, *er, zeros)
    wc = jnp.stack([W[i * _Q:(i + 1) * _Q]
                    for W in (W_i, W_ii, W_iii, W_a)
                    for i in range(4)])
    return _dense_call(h, g, wc,
                       ff_w1, ff_b1.reshape(1, INTER),
                       ff_w2, ff_b2.reshape(1, HIDDEN))


# speedup vs baseline: 2.0356x; 1.0163x over previous
import functools

import jax
import jax.numpy as jnp
from jax import lax
from jax.experimental import pallas as pl
from jax.experimental.pallas import tpu as pltpu
from jax.experimental.pallas import tpu_sc as plsc

HIDDEN = 256
INTER = 1024
N_NODES = 10000
N_EDGES = 160000
EPS = 1e-06

_Q = HIDDEN // 4
_N_SLABS = 16
_N_TILES = 16

_CHUNK = 80
_PCH = -(-N_EDGES // (_N_TILES * _CHUNK))
_EPAD = _N_TILES * _PCH * _CHUNK
_PAD_ROWS = 16
_ACC_ROWS = N_NODES + _PAD_ROWS

_RPT = 624
_REM_BASE = _RPT * _N_TILES
_REM = N_NODES - _REM_BASE
_ZR = 208


_LN_BLK = 2000


def _ln_body(x_ref, gam_ref, bet_ref, h_ref, q0_ref, q1_ref, q2_ref, q3_ref):
    x = x_ref[...]
    mu = jnp.mean(x, axis=-1, keepdims=True)
    xc = x - mu
    var = jnp.mean(xc * xc, axis=-1, keepdims=True)
    h = xc * lax.rsqrt(var + EPS) * gam_ref[...] + bet_ref[...]
    h_ref[...] = h
    q0_ref[...] = h[:, 0 * _Q:1 * _Q]
    q1_ref[...] = h[:, 1 * _Q:2 * _Q]
    q2_ref[...] = h[:, 2 * _Q:3 * _Q]
    q3_ref[...] = h[:, 3 * _Q:4 * _Q]


_ln_call = pl.pallas_call(
    _ln_body,
    grid=(N_NODES // _LN_BLK,),
    in_specs=[
        pl.BlockSpec((_LN_BLK, HIDDEN), lambda i: (i, 0)),
        pl.BlockSpec((1, HIDDEN), lambda i: (0, 0)),
        pl.BlockSpec((1, HIDDEN), lambda i: (0, 0)),
    ],
    out_specs=[pl.BlockSpec((_LN_BLK, HIDDEN), lambda i: (i, 0))] +
              [pl.BlockSpec((_LN_BLK, _Q), lambda i: (i, 0))] * 4,
    out_shape=[jax.ShapeDtypeStruct((N_NODES, HIDDEN), jnp.float32)] +
              [jax.ShapeDtypeStruct((N_NODES, _Q), jnp.float32)] * 4,
)


_sc_mesh = plsc.VectorSubcoreMesh(core_axis_name="c", subcore_axis_name="s")


@functools.partial(
    pl.kernel,
    mesh=_sc_mesh,
    out_type=jax.ShapeDtypeStruct((_N_SLABS, N_NODES, _Q), jnp.float32),
    scratch_types=[
        pltpu.VMEM((_PCH, _CHUNK), jnp.int32),
        pltpu.VMEM((_PCH, _CHUNK), jnp.int32),
        pltpu.VMEM((10, _CHUNK, _Q), jnp.float32),
        pltpu.VMEM((_ZR, _Q), jnp.float32),
        pltpu.VMEM_SHARED((_ACC_ROWS, _Q), jnp.float32),
        pltpu.SemaphoreType.DMA,
        pltpu.SemaphoreType.DMA,
    ],
    compiler_params=pltpu.CompilerParams(use_tc_tiling_on_sc=False),
)
def _sc_scatter(q0, q1, q2, q3, src0, dst0, src1, dst1, src2, dst2,
                src3, dst3, zeros_hbm, out_hbm,
                sblk, dblk, rows, zrows_v, accum, sem_g, sem_s):
    cid = lax.axis_index("c")
    sid = lax.axis_index("s")
    pltpu.sync_copy(zeros_hbm, zrows_v)
    quarters = (q0, q1, q2, q3)
    edges = ((src0, dst0), (src1, dst1), (src2, dst2), (src3, dst3))
    for k in range(4):

        @pl.when(cid == k // 2)
        def _edge_set(k=k):
            src2d, dst2d = edges[k]
            row0 = sid * _PCH
            pltpu.sync_copy(src2d.at[pl.ds(row0, _PCH)], sblk)
            pltpu.sync_copy(dst2d.at[pl.ds(row0, _PCH)], dblk)
            for q in range(4):
                s = k * 4 + q
                hq = quarters[q]
                if q == 0:
                    for i in range(5):
                        pltpu.async_copy(hq.at[sblk.at[i]], rows.at[i], sem_g)
                for t in range(_RPT // _ZR):
                    pltpu.sync_copy(
                        zrows_v, accum.at[pl.ds(sid * _RPT + t * _ZR, _ZR)])

                @pl.when(sid == 0)
                def _zrem():
                    pltpu.sync_copy(zrows_v.at[pl.ds(0, _REM)],
                                    accum.at[pl.ds(_REM_BASE, _REM)])

                plsc.subcore_barrier()

                def body(j, carry):
                    b = lax.rem(j, 10)
                    pltpu.make_async_copy(hq.at[sblk.at[j]],
                                          rows.at[b], sem_g).wait()

                    @pl.when(j >= 2)
                    def _wait_s():
                        jp = jnp.maximum(j - 2, 0)
                        pltpu.make_async_copy(rows.at[lax.rem(jp, 10)],
                                              accum.at[dblk.at[jp]],
                                              sem_s).wait()

                    @pl.when(j < _PCH - 5)
                    def _next():
                        pltpu.async_copy(hq.at[sblk.at[j + 5]],
                                         rows.at[lax.rem(j + 5, 10)], sem_g)

                    pltpu.async_copy(rows.at[b], accum.at[dblk.at[j]],
                                     sem_s, add=True)
                    return carry

                lax.fori_loop(0, _PCH, body, 0)
                pltpu.make_async_copy(rows.at[(_PCH - 2) % 10],
                                      accum.at[dblk.at[_PCH - 2]],
                                      sem_s).wait()
                pltpu.make_async_copy(rows.at[(_PCH - 1) % 10],
                                      accum.at[dblk.at[_PCH - 1]],
                                      sem_s).wait()
                if q < 3:
                    hq_next = quarters[q + 1]
                    for i in range(5):
                        pltpu.async_copy(hq_next.at[sblk.at[i]],
                                         rows.at[i], sem_g)
                plsc.subcore_barrier()
                pltpu.sync_copy(accum.at[pl.ds(sid * _RPT, _RPT)],
                                out_hbm.at[s, pl.ds(sid * _RPT, _RPT)])

                @pl.when(sid == 0)
                def _wrem():
                    pltpu.sync_copy(accum.at[pl.ds(_REM_BASE, _REM)],
                                    out_hbm.at[s, pl.ds(_REM_BASE, _REM)])


_DN_BLK = 1000


def _dense_body(h_ref, g_ref, wc_ref, w1_ref, b1_ref, w2_ref, b2_ref, o_ref):
    acc = jnp.zeros((_DN_BLK, HIDDEN), jnp.float32)
    for s in range(_N_SLABS):
        acc += jnp.dot(g_ref[s], wc_ref[s], preferred_element_type=jnp.float32)
    h2 = h_ref[...] + acc
    inter = jnp.dot(h2, w1_ref[...], preferred_element_type=jnp.float32)
    inter = inter + b1_ref[...]
    inter = jnp.where(inter >= 0, inter, 0.01 * inter)
    ff = jnp.dot(inter, w2_ref[...], preferred_element_type=jnp.float32)
    o_ref[...] = h2 + ff + b2_ref[...]


_dense_call = pl.pallas_call(
    _dense_body,
    grid=(N_NODES // _DN_BLK,),
    in_specs=[
        pl.BlockSpec((_DN_BLK, HIDDEN), lambda i: (i, 0)),
        pl.BlockSpec((_N_SLABS, _DN_BLK, _Q), lambda i: (0, i, 0)),
        pl.BlockSpec((_N_SLABS, _Q, HIDDEN), lambda i: (0, 0, 0)),
        pl.BlockSpec((HIDDEN, INTER), lambda i: (0, 0)),
        pl.BlockSpec((1, INTER), lambda i: (0, 0)),
        pl.BlockSpec((INTER, HIDDEN), lambda i: (0, 0)),
        pl.BlockSpec((1, HIDDEN), lambda i: (0, 0)),
    ],
    out_specs=pl.BlockSpec((_DN_BLK, HIDDEN), lambda i: (i, 0)),
    out_shape=jax.ShapeDtypeStruct((N_NODES, HIDDEN), jnp.float32),
)


def kernel(hidden_states, edge_index_i, edge_index_ii, edge_index_iii,
           edge_index_a, W_i, W_ii, W_iii, W_a, ln_gamma, ln_beta,
           ff_w1, ff_b1, ff_w2, ff_b2):
    h, q0, q1, q2, q3 = _ln_call(hidden_states,
                                 ln_gamma.reshape(1, HIDDEN),
                                 ln_beta.reshape(1, HIDDEN))
    npad = _EPAD - N_EDGES
    if npad:
        pad_src = jnp.zeros((npad,), jnp.int32)
        pad_dst = N_NODES + (jnp.arange(npad, dtype=jnp.int32) % _PAD_ROWS)
    er = []
    for e in (edge_index_i, edge_index_ii, edge_index_iii, edge_index_a):
        e32 = e.astype(jnp.int32)
        if npad:
            er += [jnp.concatenate([e32[0], pad_src]).reshape(-1, _CHUNK),
                   jnp.concatenate([e32[1], pad_dst]).reshape(-1, _CHUNK)]
        else:
            er += [e32[0].reshape(-1, _CHUNK), e32[1].reshape(-1, _CHUNK)]
    zeros = jnp.zeros((_ZR, _Q), jnp.float32)
    g = _sc_scatter(q0, q1, q2, q3, *er, zeros)
    wc = jnp.stack([W[i * _Q:(i + 1) * _Q]
                    for W in (W_i, W_ii, W_iii, W_a)
                    for i in range(4)])
    return _dense_call(h, g, wc,
                       ff_w1, ff_b1.reshape(1, INTER),
                       ff_w2, ff_b2.reshape(1, HIDDEN))
